# baseline (speedup 1.0000x reference)
import functools

import jax
import jax.numpy as jnp
from jax import lax
from jax.experimental import pallas as pl
from jax.experimental.pallas import tpu as pltpu
from jax.experimental.pallas import tpu_sc as plsc

N = 10000
D = 128
HD = 64
NC = 2
NS = 16
NW = NC * NS
LN = 128
CH = 80
CHS = CH * NC
EROWS = NW * CH
EPAD = EROWS * LN
NB = 4
GROUPS = CHS // NB
TRASH = N
ACC_ROWS = 10240
DEG_ROWS = 10240
BLK = 1000

_SC_PARAMS = pltpu.CompilerParams(
    needs_layout_passes=False, use_tc_tiling_on_sc=False)


def _sc_dinv(dst_t, zdeg):
    mesh = plsc.VectorSubcoreMesh(core_axis_name="c", subcore_axis_name="s")
    SL = DEG_ROWS // (NC * NS)

    @functools.partial(
        pl.kernel,
        out_type=jax.ShapeDtypeStruct((DEG_ROWS,), jnp.float32),
        mesh=mesh,
        scratch_types=[
            pltpu.VMEM((CHS, LN), jnp.int32),
            pltpu.VMEM((DEG_ROWS,), jnp.float32),
            pltpu.VMEM((NS, SL), jnp.float32),
            pltpu.VMEM((SL,), jnp.float32),
            pltpu.VMEM_SHARED((NS, DEG_ROWS), jnp.float32),
        ],
        compiler_params=_SC_PARAMS,
    )
    def k(dst_hbm, z_hbm, out_hbm, idx_v, acc_v, red_v, dinv_v, deg_s):
        c = lax.axis_index("c")
        s = lax.axis_index("s")
        pltpu.sync_copy(z_hbm, acc_v)
        pltpu.sync_copy(dst_hbm.at[pl.ds(s * CHS, CHS)], idx_v)
        ones = jnp.full((16,), 1.0, jnp.float32)

        def body(j, carry):
            for u in range(LN // 16):
                idx = idx_v[j, pl.ds(u * 16, 16)]
                plsc.addupdate_scatter(acc_v, [idx], ones)
            return carry

        lax.fori_loop(0, CHS, body, 0)
        pltpu.sync_copy(acc_v, deg_s.at[s])
        plsc.subcore_barrier()
        base = c * (NS * SL) + s * SL
        for r in range(NS):
            pltpu.sync_copy(deg_s.at[r, pl.ds(base, SL)], red_v.at[r])
        for i in range(SL // 16):
            tot = red_v[0, pl.ds(i * 16, 16)]
            for r in range(1, NS):
                tot = tot + red_v[r, pl.ds(i * 16, 16)]
            x = tot + 1.0
            iv = plsc.bitcast(x, jnp.int32)
            y = plsc.bitcast(
                jnp.int32(0x5F3759DF) - jnp.right_shift(iv, 1),
                jnp.float32)
            for _ in range(3):
                y = y * (1.5 - 0.5 * x * y * y)
            dinv_v[pl.ds(i * 16, 16)] = y
        pltpu.sync_copy(dinv_v, out_hbm.at[pl.ds(base, SL)])

    return k(dst_t, zdeg)


def _sc_agg(g, src_t, dst_t, zrows):
    mesh = plsc.VectorSubcoreMesh(core_axis_name="c", subcore_axis_name="s")

    @functools.partial(
        pl.kernel,
        out_type=jax.ShapeDtypeStruct((NC, N, HD), jnp.bfloat16),
        mesh=mesh,
        scratch_types=[
            pltpu.VMEM((CHS, LN), jnp.int32),
            pltpu.VMEM((CHS, LN), jnp.int32),
            [pltpu.VMEM((LN, HD), jnp.bfloat16) for _ in range(NB)],
            pltpu.VMEM((125, HD), jnp.bfloat16),
            pltpu.VMEM_SHARED((ACC_ROWS, HD), jnp.bfloat16),
            pltpu.VMEM_SHARED((N, HD), jnp.bfloat16),
            [pltpu.SemaphoreType.DMA for _ in range(NB)],
            [pltpu.SemaphoreType.DMA for _ in range(NB)],
        ],
        compiler_params=_SC_PARAMS,
    )
    def k(g_hbm, src_hbm, dst_hbm, z_hbm, out_hbm,
          src_v, dst_v, rows, stage_v, acc_s, tbl_s, gs, ss):
        c = lax.axis_index("c")
        s = lax.axis_index("s")
        pltpu.sync_copy(
            z_hbm, acc_s.at[pl.ds(s * (ACC_ROWS // NS), ACC_ROWS // NS)])
        pltpu.sync_copy(g_hbm.at[c, pl.ds(s * (N // NS), N // NS)],
                        tbl_s.at[pl.ds(s * (N // NS), N // NS)])
        pltpu.sync_copy(src_hbm.at[pl.ds(s * CHS, CHS)], src_v)
        pltpu.sync_copy(dst_hbm.at[pl.ds(s * CHS, CHS)], dst_v)
        plsc.subcore_barrier()
        gc = tbl_s

        for b in range(NB):
            pltpu.async_copy(gc.at[src_v.at[b]], rows[b], gs[b])

        def body(g, carry):
            j0 = g * NB
            for b in range(NB):
                j = j0 + b
                pltpu.make_async_copy(gc.at[src_v.at[j]], rows[b], gs[b]).wait()
                pltpu.async_copy(rows[b], acc_s.at[dst_v.at[j]], ss[b], add=True)

            @pl.when(g + 1 < GROUPS)
            def _():
                for b in range(NB):
                    j = j0 + b
                    pltpu.make_async_copy(
                        rows[b], acc_s.at[dst_v.at[j]], ss[b]).wait()
                    pltpu.async_copy(gc.at[src_v.at[j + NB]], rows[b], gs[b])
            return carry

        lax.fori_loop(0, GROUPS, body, 0)
        for b in range(NB):
            pltpu.make_async_copy(
                rows[b], acc_s.at[dst_v.at[CHS - NB + b]], ss[b]).wait()
        plsc.subcore_barrier()
        for i in range(5):
            r0 = s * (N // NS) + i * 125
            pltpu.sync_copy(acc_s.at[pl.ds(r0, 125)], stage_v)
            pltpu.sync_copy(stage_v, out_hbm.at[c, pl.ds(r0, 125)])

    return k(g, src_t, dst_t, zrows)


def _sparsemax3(s0, s1, s2):
    z1 = jnp.maximum(s0, jnp.maximum(s1, s2))
    z3 = jnp.minimum(s0, jnp.minimum(s1, s2))
    z2 = s0 + s1 + s2 - z1 - z3
    i2 = (1.0 + 2.0 * z2 > z1 + z2).astype(jnp.float32)
    i3 = (1.0 + 3.0 * z3 > z1 + z2 + z3).astype(jnp.float32)
    kf = 1.0 + i2 + i3
    tau = jnp.where(
        kf == 3.0, (z1 + z2 + z3 - 1.0) / 3.0,
        jnp.where(kf == 2.0, (z1 + z2 - 1.0) * 0.5, z1 - 1.0))
    w0 = jnp.maximum(s0 - tau, 0.0)
    w1 = jnp.maximum(s1 - tau, 0.0)
    w2 = jnp.maximum(s2 - tau, 0.0)
    return w0, w1, w2


def _proj_body(x_ref, w_ref, b_ref, dinv_ref, g0_ref):
    dinv = dinv_ref[...]
    x0 = jnp.dot(x_ref[...].astype(jnp.bfloat16),
                 w_ref[...].astype(jnp.bfloat16),
                 preferred_element_type=jnp.float32)
    x0 = jnp.maximum(x0 + b_ref[...], 0.0)
    g0 = (x0 * dinv).astype(jnp.bfloat16)
    g0_ref[0] = g0[:, :HD]
    g0_ref[1] = g0[:, HD:]


def _tc_proj(x, w, b2, dinv):
    return pl.pallas_call(
        _proj_body,
        grid=(N // BLK,),
        in_specs=[
            pl.BlockSpec((BLK, D), lambda i: (i, 0)),
            pl.BlockSpec((D, D), lambda i: (0, 0)),
            pl.BlockSpec((1, D), lambda i: (0, 0)),
            pl.BlockSpec((BLK, 1), lambda i: (i, 0)),
        ],
        out_specs=pl.BlockSpec((NC, BLK, HD), lambda i: (0, i, 0)),
        out_shape=jax.ShapeDtypeStruct((NC, N, HD), jnp.bfloat16),
    )(x, w, b2, dinv)


def _conv_compute(acc_ref, g_ref, dinv, w0, w1, w2, att):
    xagg = jnp.concatenate(
        [acc_ref[0].astype(jnp.float32) + g_ref[0].astype(jnp.float32),
         acc_ref[1].astype(jnp.float32) + g_ref[1].astype(jnp.float32)],
        axis=1) * dinv
    xb = xagg.astype(jnp.bfloat16)
    h0 = jnp.dot(xb, w0.astype(jnp.bfloat16),
                 preferred_element_type=jnp.float32)
    h1 = jnp.dot(xb, w1.astype(jnp.bfloat16),
                 preferred_element_type=jnp.float32)
    h2 = jnp.dot(xb, w2.astype(jnp.bfloat16),
                 preferred_element_type=jnp.float32)
    s0 = jnp.sum(h0 * att, axis=1, keepdims=True)
    s1 = jnp.sum(h1 * att, axis=1, keepdims=True)
    s2 = jnp.sum(h2 * att, axis=1, keepdims=True)
    b0, b1, b2 = _sparsemax3(s0, s1, s2)
    return jnp.maximum(b0 * h0 + b1 * h1 + b2 * h2, 0.0)


def _conv_body(acc_ref, g_ref, dinv_ref, w_ref, att_ref, out_ref):
    dinv = dinv_ref[...]
    x1 = _conv_compute(acc_ref, g_ref, dinv,
                       w_ref[0], w_ref[1], w_ref[2], att_ref[...])
    g1 = (x1 * dinv).astype(jnp.bfloat16)
    out_ref[0] = g1[:, :HD]
    out_ref[1] = g1[:, HD:]


def _tc_conv(acc, g, dinv, conv_w, att2):
    return pl.pallas_call(
        _conv_body,
        grid=(N // BLK,),
        in_specs=[
            pl.BlockSpec((NC, BLK, HD), lambda i: (0, i, 0)),
            pl.BlockSpec((NC, BLK, HD), lambda i: (0, i, 0)),
            pl.BlockSpec((BLK, 1), lambda i: (i, 0)),
            pl.BlockSpec((3, D, D), lambda i: (0, 0, 0)),
            pl.BlockSpec((1, D), lambda i: (0, 0)),
        ],
        out_specs=pl.BlockSpec((NC, BLK, HD), lambda i: (0, i, 0)),
        out_shape=jax.ShapeDtypeStruct((NC, N, HD), jnp.bfloat16),
    )(acc, g, dinv, conv_w, att2)


def _final_body(acc_ref, g_ref, dinv_ref, w_ref, att_ref,
                cw_ref, cb_ref, catt_ref, out_ref):
    dinv = dinv_ref[...]
    x2 = _conv_compute(acc_ref, g_ref, dinv,
                       w_ref[0], w_ref[1], w_ref[2], att_ref[...])
    catt = catt_ref[...]
    x2b = x2.astype(jnp.bfloat16)
    cw0 = cw_ref[0].astype(jnp.bfloat16)
    cw1 = cw_ref[1].astype(jnp.bfloat16)
    cw2 = cw_ref[2].astype(jnp.bfloat16)
    h0 = jnp.dot(x2b, cw0, preferred_element_type=jnp.float32) + cb_ref[0:1, :]
    h1 = jnp.dot(x2b, cw1, preferred_element_type=jnp.float32) + cb_ref[1:2, :]
    h2 = jnp.dot(x2b, cw2, preferred_element_type=jnp.float32) + cb_ref[2:3, :]
    s0 = jnp.sum(h0 * catt, axis=1, keepdims=True)
    s1 = jnp.sum(h1 * catt, axis=1, keepdims=True)
    s2 = jnp.sum(h2 * catt, axis=1, keepdims=True)
    b0, b1, b2 = _sparsemax3(s0, s1, s2)
    out_ref[...] = b0 * h0 + b1 * h1 + b2 * h2


def _tc_final(acc, g, dinv, conv_w, att2, cwp, cbp, cattp):
    return pl.pallas_call(
        _final_body,
        grid=(N // BLK,),
        in_specs=[
            pl.BlockSpec((NC, BLK, HD), lambda i: (0, i, 0)),
            pl.BlockSpec((NC, BLK, HD), lambda i: (0, i, 0)),
            pl.BlockSpec((BLK, 1), lambda i: (i, 0)),
            pl.BlockSpec((3, D, D), lambda i: (0, 0, 0)),
            pl.BlockSpec((1, D), lambda i: (0, 0)),
            pl.BlockSpec((3, D, D), lambda i: (0, 0, 0)),
            pl.BlockSpec((3, D), lambda i: (0, 0)),
            pl.BlockSpec((1, D), lambda i: (0, 0)),
        ],
        out_specs=pl.BlockSpec((BLK, D), lambda i: (i, 0)),
        out_shape=jax.ShapeDtypeStruct((N, D), jnp.float32),
    )(acc, g, dinv, conv_w, att2, cwp, cbp, cattp)


def kernel(x, edge_index, W_lin, b_lin, conv1_w, conv1_att, conv2_w,
           conv2_att, cls_w, cls_b, cls_att):
    src = edge_index[0]
    dst = edge_index[1]
    e = src.shape[0]
    pad = jnp.arange(EPAD - e, dtype=jnp.int32)
    src_t = jnp.concatenate([src, pad % N]).reshape(EROWS, LN)
    dst_t = jnp.concatenate(
        [dst, TRASH + pad % (ACC_ROWS - N)]).reshape(EROWS, LN)
    zdeg = jnp.zeros((DEG_ROWS,), jnp.float32)
    zrows = jnp.zeros((ACC_ROWS // NS, HD), jnp.bfloat16)

    dinv = _sc_dinv(dst_t, zdeg)[:N].reshape(N, 1)

    g0 = _tc_proj(x, W_lin, b_lin.reshape(1, D), dinv)
    acc1 = _sc_agg(g0, src_t, dst_t, zrows)
    g1 = _tc_conv(acc1, g0, dinv, conv1_w, conv1_att[:, 0].reshape(1, D))
    acc2 = _sc_agg(g1, src_t, dst_t, zrows)

    c = cls_w.shape[2]
    cwp = jnp.zeros((3, D, D), jnp.float32).at[:, :, :c].set(cls_w)
    cbp = jnp.zeros((3, D), jnp.float32).at[:, :c].set(cls_b)
    cattp = jnp.zeros((1, D), jnp.float32).at[0, :c].set(cls_att[:, 0])
    out = _tc_final(acc2, g1, dinv, conv2_w, conv2_att[:, 0].reshape(1, D),
                    cwp, cbp, cattp)
    return out[:, :c]


# speedup vs baseline: 1.0008x; 1.0008x over previous
import functools

import jax
import jax.numpy as jnp
from jax import lax
from jax.experimental import pallas as pl
from jax.experimental.pallas import tpu as pltpu
from jax.experimental.pallas import tpu_sc as plsc

N = 10000
D = 128
HD = 64
NC = 2
NS = 16
NW = NC * NS
LN = 128
CH = 80
CHS = CH * NC
EROWS = NW * CH
EPAD = EROWS * LN
NB = 4
GROUPS = CHS // NB
TRASH = N
ACC_ROWS = 10240
DEG_ROWS = 10240
BLK = 1000

_SC_PARAMS = pltpu.CompilerParams(
    needs_layout_passes=False, use_tc_tiling_on_sc=False)


def _sc_dinv(dst_t, zdeg):
    mesh = plsc.VectorSubcoreMesh(core_axis_name="c", subcore_axis_name="s")
    SL = DEG_ROWS // (NC * NS)

    @functools.partial(
        pl.kernel,
        out_type=jax.ShapeDtypeStruct((DEG_ROWS,), jnp.float32),
        mesh=mesh,
        scratch_types=[
            pltpu.VMEM((CHS, LN), jnp.int32),
            pltpu.VMEM((DEG_ROWS,), jnp.float32),
            pltpu.VMEM((NS, SL), jnp.float32),
            pltpu.VMEM((SL,), jnp.float32),
            pltpu.VMEM_SHARED((NS, DEG_ROWS), jnp.float32),
        ],
        compiler_params=_SC_PARAMS,
    )
    def k(dst_hbm, z_hbm, out_hbm, idx_v, acc_v, red_v, dinv_v, deg_s):
        c = lax.axis_index("c")
        s = lax.axis_index("s")
        pltpu.sync_copy(z_hbm, acc_v)
        pltpu.sync_copy(dst_hbm.at[pl.ds(s * CHS, CHS)], idx_v)
        ones = jnp.full((16,), 1.0, jnp.float32)

        def body(j, carry):
            for u in range(LN // 16):
                idx = idx_v[j, pl.ds(u * 16, 16)]
                plsc.addupdate_scatter(acc_v, [idx], ones)
            return carry

        lax.fori_loop(0, CHS, body, 0)
        pltpu.sync_copy(acc_v, deg_s.at[s])
        plsc.subcore_barrier()
        base = c * (NS * SL) + s * SL
        for r in range(NS):
            pltpu.sync_copy(deg_s.at[r, pl.ds(base, SL)], red_v.at[r])
        for i in range(SL // 16):
            tot = red_v[0, pl.ds(i * 16, 16)]
            for r in range(1, NS):
                tot = tot + red_v[r, pl.ds(i * 16, 16)]
            x = tot + 1.0
            iv = plsc.bitcast(x, jnp.int32)
            y = plsc.bitcast(
                jnp.int32(0x5F3759DF) - jnp.right_shift(iv, 1),
                jnp.float32)
            for _ in range(3):
                y = y * (1.5 - 0.5 * x * y * y)
            dinv_v[pl.ds(i * 16, 16)] = y
        pltpu.sync_copy(dinv_v, out_hbm.at[pl.ds(base, SL)])

    return k(dst_t, zdeg)


def _sc_agg(g, src_t, dst_t, zrows):
    mesh = plsc.VectorSubcoreMesh(core_axis_name="c", subcore_axis_name="s")

    @functools.partial(
        pl.kernel,
        out_type=jax.ShapeDtypeStruct((NC, N, HD), jnp.bfloat16),
        mesh=mesh,
        scratch_types=[
            pltpu.VMEM((CHS, LN), jnp.int32),
            pltpu.VMEM((CHS, LN), jnp.int32),
            [pltpu.VMEM((LN, HD), jnp.bfloat16) for _ in range(NB)],
            pltpu.VMEM((125, HD), jnp.bfloat16),
            pltpu.VMEM_SHARED((ACC_ROWS, HD), jnp.bfloat16),
            pltpu.VMEM_SHARED((N, HD), jnp.bfloat16),
            [pltpu.SemaphoreType.DMA for _ in range(NB)],
            [pltpu.SemaphoreType.DMA for _ in range(NB)],
        ],
        compiler_params=_SC_PARAMS,
    )
    def k(g_hbm, src_hbm, dst_hbm, z_hbm, out_hbm,
          src_v, dst_v, rows, stage_v, acc_s, tbl_s, gs, ss):
        c = lax.axis_index("c")
        s = lax.axis_index("s")
        pltpu.sync_copy(
            z_hbm, acc_s.at[pl.ds(s * (ACC_ROWS // NS), ACC_ROWS // NS)])
        pltpu.sync_copy(g_hbm.at[c, pl.ds(s * (N // NS), N // NS)],
                        tbl_s.at[pl.ds(s * (N // NS), N // NS)])
        pltpu.sync_copy(src_hbm.at[pl.ds(s * CHS, CHS)], src_v)
        pltpu.sync_copy(dst_hbm.at[pl.ds(s * CHS, CHS)], dst_v)
        plsc.subcore_barrier()
        gc = tbl_s

        for b in range(NB):
            pltpu.async_copy(gc.at[src_v.at[b]], rows[b], gs[b])

        def body(g, carry):
            j0 = g * NB
            for b in range(NB):
                j = j0 + b
                pltpu.make_async_copy(gc.at[src_v.at[j]], rows[b], gs[b]).wait()
                pltpu.async_copy(rows[b], acc_s.at[dst_v.at[j]], ss[b], add=True)

            @pl.when(g + 1 < GROUPS)
            def _():
                for b in range(NB):
                    j = j0 + b
                    pltpu.make_async_copy(
                        rows[b], acc_s.at[dst_v.at[j]], ss[b]).wait()
                    pltpu.async_copy(gc.at[src_v.at[j + NB]], rows[b], gs[b])
            return carry

        lax.fori_loop(0, GROUPS, body, 0)
        for b in range(NB):
            pltpu.make_async_copy(
                rows[b], acc_s.at[dst_v.at[CHS - NB + b]], ss[b]).wait()
        plsc.subcore_barrier()
        for i in range(5):
            r0 = s * (N // NS) + i * 125
            pltpu.sync_copy(acc_s.at[pl.ds(r0, 125)], stage_v)
            pltpu.sync_copy(stage_v, out_hbm.at[c, pl.ds(r0, 125)])

    return k(g, src_t, dst_t, zrows)


def _sparsemax3(s0, s1, s2):
    z1 = jnp.maximum(s0, jnp.maximum(s1, s2))
    z3 = jnp.minimum(s0, jnp.minimum(s1, s2))
    z2 = s0 + s1 + s2 - z1 - z3
    i2 = (1.0 + 2.0 * z2 > z1 + z2).astype(jnp.float32)
    i3 = (1.0 + 3.0 * z3 > z1 + z2 + z3).astype(jnp.float32)
    kf = 1.0 + i2 + i3
    tau = jnp.where(
        kf == 3.0, (z1 + z2 + z3 - 1.0) / 3.0,
        jnp.where(kf == 2.0, (z1 + z2 - 1.0) * 0.5, z1 - 1.0))
    w0 = jnp.maximum(s0 - tau, 0.0)
    w1 = jnp.maximum(s1 - tau, 0.0)
    w2 = jnp.maximum(s2 - tau, 0.0)
    return w0, w1, w2


def _proj_body(x_ref, w_ref, b_ref, dinv_ref, g0_ref):
    dinv = dinv_ref[...]
    x0 = jnp.dot(x_ref[...].astype(jnp.bfloat16),
                 w_ref[...].astype(jnp.bfloat16),
                 preferred_element_type=jnp.float32)
    x0 = jnp.maximum(x0 + b_ref[...], 0.0)
    g0 = (x0 * dinv).astype(jnp.bfloat16)
    g0_ref[0] = g0[:, :HD]
    g0_ref[1] = g0[:, HD:]


def _tc_proj(x, w, b2, dinv):
    return pl.pallas_call(
        _proj_body,
        grid=(N // BLK,),
        in_specs=[
            pl.BlockSpec((BLK, D), lambda i: (i, 0)),
            pl.BlockSpec((D, D), lambda i: (0, 0)),
            pl.BlockSpec((1, D), lambda i: (0, 0)),
            pl.BlockSpec((BLK, 1), lambda i: (i, 0)),
        ],
        out_specs=pl.BlockSpec((NC, BLK, HD), lambda i: (0, i, 0)),
        out_shape=jax.ShapeDtypeStruct((NC, N, HD), jnp.bfloat16),
    )(x, w, b2, dinv)


def _conv_compute(acc_ref, g_ref, dinv, w0, w1, w2, att):
    xagg = jnp.concatenate(
        [acc_ref[0].astype(jnp.float32) + g_ref[0].astype(jnp.float32),
         acc_ref[1].astype(jnp.float32) + g_ref[1].astype(jnp.float32)],
        axis=1) * dinv
    xb = xagg.astype(jnp.bfloat16)
    h0 = jnp.dot(xb, w0.astype(jnp.bfloat16),
                 preferred_element_type=jnp.float32)
    h1 = jnp.dot(xb, w1.astype(jnp.bfloat16),
                 preferred_element_type=jnp.float32)
    h2 = jnp.dot(xb, w2.astype(jnp.bfloat16),
                 preferred_element_type=jnp.float32)
    s0 = jnp.sum(h0 * att, axis=1, keepdims=True)
    s1 = jnp.sum(h1 * att, axis=1, keepdims=True)
    s2 = jnp.sum(h2 * att, axis=1, keepdims=True)
    b0, b1, b2 = _sparsemax3(s0, s1, s2)
    return jnp.maximum(b0 * h0 + b1 * h1 + b2 * h2, 0.0)


def _conv_body(acc_ref, g_ref, dinv_ref, w_ref, att_ref, out_ref):
    dinv = dinv_ref[...]
    x1 = _conv_compute(acc_ref, g_ref, dinv,
                       w_ref[0], w_ref[1], w_ref[2], att_ref[...])
    g1 = (x1 * dinv).astype(jnp.bfloat16)
    out_ref[0] = g1[:, :HD]
    out_ref[1] = g1[:, HD:]


def _tc_conv(acc, g, dinv, conv_w, att2):
    return pl.pallas_call(
        _conv_body,
        grid=(N // BLK,),
        in_specs=[
            pl.BlockSpec((NC, BLK, HD), lambda i: (0, i, 0)),
            pl.BlockSpec((NC, BLK, HD), lambda i: (0, i, 0)),
            pl.BlockSpec((BLK, 1), lambda i: (i, 0)),
            pl.BlockSpec((3, D, D), lambda i: (0, 0, 0)),
            pl.BlockSpec((1, D), lambda i: (0, 0)),
        ],
        out_specs=pl.BlockSpec((NC, BLK, HD), lambda i: (0, i, 0)),
        out_shape=jax.ShapeDtypeStruct((NC, N, HD), jnp.bfloat16),
    )(acc, g, dinv, conv_w, att2)


def _final_body(acc_ref, g_ref, dinv_ref, w_ref, att_ref,
                cw_ref, cb_ref, catt_ref, out_ref):
    dinv = dinv_ref[...]
    x2 = _conv_compute(acc_ref, g_ref, dinv,
                       w_ref[0], w_ref[1], w_ref[2], att_ref[...])
    catt = catt_ref[...]
    x2b = x2.astype(jnp.bfloat16)
    cw0 = cw_ref[0].astype(jnp.bfloat16)
    cw1 = cw_ref[1].astype(jnp.bfloat16)
    cw2 = cw_ref[2].astype(jnp.bfloat16)
    h0 = jnp.dot(x2b, cw0, preferred_element_type=jnp.float32) + cb_ref[0:1, :]
    h1 = jnp.dot(x2b, cw1, preferred_element_type=jnp.float32) + cb_ref[1:2, :]
    h2 = jnp.dot(x2b, cw2, preferred_element_type=jnp.float32) + cb_ref[2:3, :]
    s0 = jnp.sum(h0 * catt, axis=1, keepdims=True)
    s1 = jnp.sum(h1 * catt, axis=1, keepdims=True)
    s2 = jnp.sum(h2 * catt, axis=1, keepdims=True)
    b0, b1, b2 = _sparsemax3(s0, s1, s2)
    out_ref[...] = b0 * h0 + b1 * h1 + b2 * h2


def _tc_final(acc, g, dinv, conv_w, att2, cwp, cbp, cattp):
    return pl.pallas_call(
        _final_body,
        grid=(N // BLK,),
        in_specs=[
            pl.BlockSpec((NC, BLK, HD), lambda i: (0, i, 0)),
            pl.BlockSpec((NC, BLK, HD), lambda i: (0, i, 0)),
            pl.BlockSpec((BLK, 1), lambda i: (i, 0)),
            pl.BlockSpec((3, D, D), lambda i: (0, 0, 0)),
            pl.BlockSpec((1, D), lambda i: (0, 0)),
            pl.BlockSpec((3, D, D), lambda i: (0, 0, 0)),
            pl.BlockSpec((3, D), lambda i: (0, 0)),
            pl.BlockSpec((1, D), lambda i: (0, 0)),
        ],
        out_specs=pl.BlockSpec((BLK, D), lambda i: (i, 0)),
        out_shape=jax.ShapeDtypeStruct((N, D), jnp.float32),
    )(acc, g, dinv, conv_w, att2, cwp, cbp, cattp)


def kernel(x, edge_index, W_lin, b_lin, conv1_w, conv1_att, conv2_w,
           conv2_att, cls_w, cls_b, cls_att):
    e = edge_index.shape[1]
    pad = jnp.arange(EPAD - e, dtype=jnp.int32)
    src_t = jnp.concatenate(
        [edge_index[0].reshape(e // LN, LN), (pad % N).reshape(-1, LN)])
    dst_t = jnp.concatenate(
        [edge_index[1].reshape(e // LN, LN),
         (TRASH + pad % (ACC_ROWS - N)).reshape(-1, LN)])
    zdeg = jnp.zeros((DEG_ROWS,), jnp.float32)
    zrows = jnp.zeros((ACC_ROWS // NS, HD), jnp.bfloat16)

    dinv = _sc_dinv(dst_t, zdeg)[:N].reshape(N, 1)

    g0 = _tc_proj(x, W_lin, b_lin.reshape(1, D), dinv)
    acc1 = _sc_agg(g0, src_t, dst_t, zrows)
    g1 = _tc_conv(acc1, g0, dinv, conv1_w, conv1_att[:, 0].reshape(1, D))
    acc2 = _sc_agg(g1, src_t, dst_t, zrows)

    c = cls_w.shape[2]
    cwp = jnp.zeros((3, D, D), jnp.float32).at[:, :, :c].set(cls_w)
    cbp = jnp.zeros((3, D), jnp.float32).at[:, :c].set(cls_b)
    cattp = jnp.zeros((1, D), jnp.float32).at[0, :c].set(cls_att[:, 0])
    out = _tc_final(acc2, g1, dinv, conv2_w, conv2_att[:, 0].reshape(1, D),
                    cwp, cbp, cattp)
    return out[:, :c]


# speedup vs baseline: 1.0234x; 1.0226x over previous
import functools

import jax
import jax.numpy as jnp
from jax import lax
from jax.experimental import pallas as pl
from jax.experimental.pallas import tpu as pltpu
from jax.experimental.pallas import tpu_sc as plsc

N = 10000
D = 128
HD = 64
NC = 2
NS = 16
NW = NC * NS
LN = 128
CH = 80
CHS = CH * NC
EROWS = NW * CH
EPAD = EROWS * LN
MROWS = 2500
TROWS = EROWS - NW * CH // NC * NC
TAILR = EROWS - MROWS
NB = 4
GROUPS = CHS // NB
TRASH = N
ACC_ROWS = 10240
DEG_ROWS = 10240
BLK = 1000

_SC_PARAMS = pltpu.CompilerParams(
    needs_layout_passes=False, use_tc_tiling_on_sc=False)


def _load_idx(ei_hbm, row, tail_hbm, s, idx_v):
    @pl.when(s < NS - 1)
    def _full():
        pltpu.sync_copy(ei_hbm.at[row, pl.ds(s * CHS, CHS)], idx_v)

    @pl.when(s == NS - 1)
    def _tail():
        pltpu.sync_copy(ei_hbm.at[row, pl.ds(s * CHS, MROWS - (NS - 1) * CHS)],
                        idx_v.at[pl.ds(0, MROWS - (NS - 1) * CHS)])
        pltpu.sync_copy(tail_hbm,
                        idx_v.at[pl.ds(MROWS - (NS - 1) * CHS, TAILR)])


def _sc_dinv(ei3, tail_dst, zdeg):
    mesh = plsc.VectorSubcoreMesh(core_axis_name="c", subcore_axis_name="s")
    SL = DEG_ROWS // (NC * NS)

    @functools.partial(
        pl.kernel,
        out_type=jax.ShapeDtypeStruct((DEG_ROWS,), jnp.float32),
        mesh=mesh,
        scratch_types=[
            pltpu.VMEM((CHS, LN), jnp.int32),
            pltpu.VMEM((DEG_ROWS,), jnp.float32),
            pltpu.VMEM((NS, SL), jnp.float32),
            pltpu.VMEM((SL,), jnp.float32),
            pltpu.VMEM_SHARED((NS, DEG_ROWS), jnp.float32),
        ],
        compiler_params=_SC_PARAMS,
    )
    def k(ei_hbm, td_hbm, z_hbm, out_hbm, idx_v, acc_v, red_v, dinv_v, deg_s):
        c = lax.axis_index("c")
        s = lax.axis_index("s")
        pltpu.sync_copy(z_hbm, acc_v)
        _load_idx(ei_hbm, 1, td_hbm, s, idx_v)
        ones = jnp.full((16,), 1.0, jnp.float32)

        def body(j, carry):
            for u in range(LN // 16):
                idx = idx_v[j, pl.ds(u * 16, 16)]
                plsc.addupdate_scatter(acc_v, [idx], ones)
            return carry

        lax.fori_loop(0, CHS, body, 0)
        pltpu.sync_copy(acc_v, deg_s.at[s])
        plsc.subcore_barrier()
        base = c * (NS * SL) + s * SL
        for r in range(NS):
            pltpu.sync_copy(deg_s.at[r, pl.ds(base, SL)], red_v.at[r])
        for i in range(SL // 16):
            tot = red_v[0, pl.ds(i * 16, 16)]
            for r in range(1, NS):
                tot = tot + red_v[r, pl.ds(i * 16, 16)]
            x = tot + 1.0
            iv = plsc.bitcast(x, jnp.int32)
            y = plsc.bitcast(
                jnp.int32(0x5F3759DF) - jnp.right_shift(iv, 1),
                jnp.float32)
            for _ in range(3):
                y = y * (1.5 - 0.5 * x * y * y)
            dinv_v[pl.ds(i * 16, 16)] = y
        pltpu.sync_copy(dinv_v, out_hbm.at[pl.ds(base, SL)])

    return k(ei3, tail_dst, zdeg)


def _sc_agg(g, ei3, tail_src, tail_dst, zrows):
    mesh = plsc.VectorSubcoreMesh(core_axis_name="c", subcore_axis_name="s")

    @functools.partial(
        pl.kernel,
        out_type=jax.ShapeDtypeStruct((NC, N, HD), jnp.bfloat16),
        mesh=mesh,
        scratch_types=[
            pltpu.VMEM((CHS, LN), jnp.int32),
            pltpu.VMEM((CHS, LN), jnp.int32),
            [pltpu.VMEM((LN, HD), jnp.bfloat16) for _ in range(NB)],
            pltpu.VMEM((125, HD), jnp.bfloat16),
            pltpu.VMEM_SHARED((ACC_ROWS, HD), jnp.bfloat16),
            pltpu.VMEM_SHARED((N, HD), jnp.bfloat16),
            [pltpu.SemaphoreType.DMA for _ in range(NB)],
            [pltpu.SemaphoreType.DMA for _ in range(NB)],
        ],
        compiler_params=_SC_PARAMS,
    )
    def k(g_hbm, ei_hbm, ts_hbm, td_hbm, z_hbm, out_hbm,
          src_v, dst_v, rows, stage_v, acc_s, tbl_s, gs, ss):
        c = lax.axis_index("c")
        s = lax.axis_index("s")
        pltpu.sync_copy(
            z_hbm, acc_s.at[pl.ds(s * (ACC_ROWS // NS), ACC_ROWS // NS)])
        pltpu.sync_copy(g_hbm.at[c, pl.ds(s * (N // NS), N // NS)],
                        tbl_s.at[pl.ds(s * (N // NS), N // NS)])
        _load_idx(ei_hbm, 0, ts_hbm, s, src_v)
        _load_idx(ei_hbm, 1, td_hbm, s, dst_v)
        plsc.subcore_barrier()
        gc = tbl_s

        for b in range(NB):
            pltpu.async_copy(gc.at[src_v.at[b]], rows[b], gs[b])

        def body(g, carry):
            j0 = g * NB
            for b in range(NB):
                j = j0 + b
                pltpu.make_async_copy(gc.at[src_v.at[j]], rows[b], gs[b]).wait()
                pltpu.async_copy(rows[b], acc_s.at[dst_v.at[j]], ss[b], add=True)

            @pl.when(g + 1 < GROUPS)
            def _():
                for b in range(NB):
                    j = j0 + b
                    pltpu.make_async_copy(
                        rows[b], acc_s.at[dst_v.at[j]], ss[b]).wait()
                    pltpu.async_copy(gc.at[src_v.at[j + NB]], rows[b], gs[b])
            return carry

        lax.fori_loop(0, GROUPS, body, 0)
        for b in range(NB):
            pltpu.make_async_copy(
                rows[b], acc_s.at[dst_v.at[CHS - NB + b]], ss[b]).wait()
        plsc.subcore_barrier()
        for i in range(5):
            r0 = s * (N // NS) + i * 125
            pltpu.sync_copy(acc_s.at[pl.ds(r0, 125)], stage_v)
            pltpu.sync_copy(stage_v, out_hbm.at[c, pl.ds(r0, 125)])

    return k(g, ei3, tail_src, tail_dst, zrows)


def _sparsemax3(s0, s1, s2):
    z1 = jnp.maximum(s0, jnp.maximum(s1, s2))
    z3 = jnp.minimum(s0, jnp.minimum(s1, s2))
    z2 = s0 + s1 + s2 - z1 - z3
    i2 = (1.0 + 2.0 * z2 > z1 + z2).astype(jnp.float32)
    i3 = (1.0 + 3.0 * z3 > z1 + z2 + z3).astype(jnp.float32)
    kf = 1.0 + i2 + i3
    tau = jnp.where(
        kf == 3.0, (z1 + z2 + z3 - 1.0) / 3.0,
        jnp.where(kf == 2.0, (z1 + z2 - 1.0) * 0.5, z1 - 1.0))
    w0 = jnp.maximum(s0 - tau, 0.0)
    w1 = jnp.maximum(s1 - tau, 0.0)
    w2 = jnp.maximum(s2 - tau, 0.0)
    return w0, w1, w2


def _proj_body(x_ref, w_ref, b_ref, dinv_ref, g0_ref):
    dinv = dinv_ref[...]
    x0 = jnp.dot(x_ref[...].astype(jnp.bfloat16),
                 w_ref[...].astype(jnp.bfloat16),
                 preferred_element_type=jnp.float32)
    x0 = jnp.maximum(x0 + b_ref[...], 0.0)
    g0 = (x0 * dinv).astype(jnp.bfloat16)
    g0_ref[0] = g0[:, :HD]
    g0_ref[1] = g0[:, HD:]


def _tc_proj(x, w, b2, dinv):
    return pl.pallas_call(
        _proj_body,
        grid=(N // BLK,),
        in_specs=[
            pl.BlockSpec((BLK, D), lambda i: (i, 0)),
            pl.BlockSpec((D, D), lambda i: (0, 0)),
            pl.BlockSpec((1, D), lambda i: (0, 0)),
            pl.BlockSpec((BLK, 1), lambda i: (i, 0)),
        ],
        out_specs=pl.BlockSpec((NC, BLK, HD), lambda i: (0, i, 0)),
        out_shape=jax.ShapeDtypeStruct((NC, N, HD), jnp.bfloat16),
    )(x, w, b2, dinv)


def _conv_compute(acc_ref, g_ref, dinv, w0, w1, w2, att):
    xagg = jnp.concatenate(
        [acc_ref[0].astype(jnp.float32) + g_ref[0].astype(jnp.float32),
         acc_ref[1].astype(jnp.float32) + g_ref[1].astype(jnp.float32)],
        axis=1) * dinv
    xb = xagg.astype(jnp.bfloat16)
    h0 = jnp.dot(xb, w0.astype(jnp.bfloat16),
                 preferred_element_type=jnp.float32)
    h1 = jnp.dot(xb, w1.astype(jnp.bfloat16),
                 preferred_element_type=jnp.float32)
    h2 = jnp.dot(xb, w2.astype(jnp.bfloat16),
                 preferred_element_type=jnp.float32)
    s0 = jnp.sum(h0 * att, axis=1, keepdims=True)
    s1 = jnp.sum(h1 * att, axis=1, keepdims=True)
    s2 = jnp.sum(h2 * att, axis=1, keepdims=True)
    b0, b1, b2 = _sparsemax3(s0, s1, s2)
    return jnp.maximum(b0 * h0 + b1 * h1 + b2 * h2, 0.0)


def _conv_body(acc_ref, g_ref, dinv_ref, w_ref, att_ref, out_ref):
    dinv = dinv_ref[...]
    x1 = _conv_compute(acc_ref, g_ref, dinv,
                       w_ref[0], w_ref[1], w_ref[2], att_ref[...])
    g1 = (x1 * dinv).astype(jnp.bfloat16)
    out_ref[0] = g1[:, :HD]
    out_ref[1] = g1[:, HD:]


def _tc_conv(acc, g, dinv, conv_w, att2):
    return pl.pallas_call(
        _conv_body,
        grid=(N // BLK,),
        in_specs=[
            pl.BlockSpec((NC, BLK, HD), lambda i: (0, i, 0)),
            pl.BlockSpec((NC, BLK, HD), lambda i: (0, i, 0)),
            pl.BlockSpec((BLK, 1), lambda i: (i, 0)),
            pl.BlockSpec((3, D, D), lambda i: (0, 0, 0)),
            pl.BlockSpec((1, D), lambda i: (0, 0)),
        ],
        out_specs=pl.BlockSpec((NC, BLK, HD), lambda i: (0, i, 0)),
        out_shape=jax.ShapeDtypeStruct((NC, N, HD), jnp.bfloat16),
    )(acc, g, dinv, conv_w, att2)


def _final_body(acc_ref, g_ref, dinv_ref, w_ref, att_ref,
                cw_ref, cb_ref, catt_ref, out_ref):
    dinv = dinv_ref[...]
    x2 = _conv_compute(acc_ref, g_ref, dinv,
                       w_ref[0], w_ref[1], w_ref[2], att_ref[...])
    catt = catt_ref[...]
    x2b = x2.astype(jnp.bfloat16)
    cw0 = cw_ref[0].astype(jnp.bfloat16)
    cw1 = cw_ref[1].astype(jnp.bfloat16)
    cw2 = cw_ref[2].astype(jnp.bfloat16)
    h0 = jnp.dot(x2b, cw0, preferred_element_type=jnp.float32) + cb_ref[0:1, :]
    h1 = jnp.dot(x2b, cw1, preferred_element_type=jnp.float32) + cb_ref[1:2, :]
    h2 = jnp.dot(x2b, cw2, preferred_element_type=jnp.float32) + cb_ref[2:3, :]
    s0 = jnp.sum(h0 * catt, axis=1, keepdims=True)
    s1 = jnp.sum(h1 * catt, axis=1, keepdims=True)
    s2 = jnp.sum(h2 * catt, axis=1, keepdims=True)
    b0, b1, b2 = _sparsemax3(s0, s1, s2)
    out_ref[...] = b0 * h0 + b1 * h1 + b2 * h2


def _tc_final(acc, g, dinv, conv_w, att2, cwp, cbp, cattp):
    return pl.pallas_call(
        _final_body,
        grid=(N // BLK,),
        in_specs=[
            pl.BlockSpec((NC, BLK, HD), lambda i: (0, i, 0)),
            pl.BlockSpec((NC, BLK, HD), lambda i: (0, i, 0)),
            pl.BlockSpec((BLK, 1), lambda i: (i, 0)),
            pl.BlockSpec((3, D, D), lambda i: (0, 0, 0)),
            pl.BlockSpec((1, D), lambda i: (0, 0)),
            pl.BlockSpec((3, D, D), lambda i: (0, 0, 0)),
            pl.BlockSpec((3, D), lambda i: (0, 0)),
            pl.BlockSpec((1, D), lambda i: (0, 0)),
        ],
        out_specs=pl.BlockSpec((BLK, D), lambda i: (i, 0)),
        out_shape=jax.ShapeDtypeStruct((N, D), jnp.float32),
    )(acc, g, dinv, conv_w, att2, cwp, cbp, cattp)


def kernel(x, edge_index, W_lin, b_lin, conv1_w, conv1_att, conv2_w,
           conv2_att, cls_w, cls_b, cls_att):
    e = edge_index.shape[1]
    ei3 = edge_index.reshape(2, e // LN, LN)
    pad = jnp.arange(EPAD - e, dtype=jnp.int32)
    tail_src = (pad % N).reshape(TAILR, LN)
    tail_dst = (TRASH + pad % (ACC_ROWS - N)).reshape(TAILR, LN)
    zdeg = jnp.zeros((DEG_ROWS,), jnp.float32)
    zrows = jnp.zeros((ACC_ROWS // NS, HD), jnp.bfloat16)

    dinv = _sc_dinv(ei3, tail_dst, zdeg)[:N].reshape(N, 1)

    g0 = _tc_proj(x, W_lin, b_lin.reshape(1, D), dinv)
    acc1 = _sc_agg(g0, ei3, tail_src, tail_dst, zrows)
    g1 = _tc_conv(acc1, g0, dinv, conv1_w, conv1_att[:, 0].reshape(1, D))
    acc2 = _sc_agg(g1, ei3, tail_src, tail_dst, zrows)

    c = cls_w.shape[2]
    cwp = jnp.zeros((3, D, D), jnp.float32).at[:, :, :c].set(cls_w)
    cbp = jnp.zeros((3, D), jnp.float32).at[:, :c].set(cls_b)
    cattp = jnp.zeros((1, D), jnp.float32).at[0, :c].set(cls_att[:, 0])
    out = _tc_final(acc2, g1, dinv, conv2_w, conv2_att[:, 0].reshape(1, D),
                    cwp, cbp, cattp)
    return out[:, :c]


# speedup vs baseline: 1.0245x; 1.0011x over previous
import functools

import jax
import jax.numpy as jnp
from jax import lax
from jax.experimental import pallas as pl
from jax.experimental.pallas import tpu as pltpu
from jax.experimental.pallas import tpu_sc as plsc

N = 10000
D = 128
HD = 64
NC = 2
NS = 16
NW = NC * NS
LN = 128
CH = 80
CHS = CH * NC
EROWS = NW * CH
EPAD = EROWS * LN
MROWS = 2500
TAILR = EROWS - MROWS
NB = 4
GROUPS = CHS // NB
TRASH = N
ACC_ROWS = 10240
DEG_ROWS = 10240
BLK = 1000

_SC_PARAMS = pltpu.CompilerParams(
    needs_layout_passes=False, use_tc_tiling_on_sc=False)


def _load_idx(ei_hbm, row, tail_hbm, s, idx_v):
    @pl.when(s < NS - 1)
    def _full():
        pltpu.sync_copy(ei_hbm.at[row, pl.ds(s * CHS, CHS)], idx_v)

    @pl.when(s == NS - 1)
    def _tail():
        pltpu.sync_copy(ei_hbm.at[row, pl.ds(s * CHS, MROWS - (NS - 1) * CHS)],
                        idx_v.at[pl.ds(0, MROWS - (NS - 1) * CHS)])
        pltpu.sync_copy(tail_hbm,
                        idx_v.at[pl.ds(MROWS - (NS - 1) * CHS, TAILR)])


def _sc_dinv(ei3, tail_dst, zdeg):
    mesh = plsc.VectorSubcoreMesh(core_axis_name="c", subcore_axis_name="s")
    SL = DEG_ROWS // (NC * NS)

    @functools.partial(
        pl.kernel,
        out_type=jax.ShapeDtypeStruct((DEG_ROWS,), jnp.float32),
        mesh=mesh,
        scratch_types=[
            pltpu.VMEM((CHS, LN), jnp.int32),
            pltpu.VMEM((DEG_ROWS,), jnp.float32),
            pltpu.VMEM((NS, SL), jnp.float32),
            pltpu.VMEM((SL,), jnp.float32),
            pltpu.VMEM_SHARED((NS, DEG_ROWS), jnp.float32),
        ],
        compiler_params=_SC_PARAMS,
    )
    def k(ei_hbm, td_hbm, z_hbm, out_hbm, idx_v, acc_v, red_v, dinv_v, deg_s):
        c = lax.axis_index("c")
        s = lax.axis_index("s")
        pltpu.sync_copy(z_hbm, acc_v)
        _load_idx(ei_hbm, 1, td_hbm, s, idx_v)
        ones = jnp.full((16,), 1.0, jnp.float32)

        def body(j, carry):
            for u in range(LN // 16):
                idx = idx_v[j, pl.ds(u * 16, 16)]
                plsc.addupdate_scatter(acc_v, [idx], ones)
            return carry

        lax.fori_loop(0, CHS, body, 0)
        pltpu.sync_copy(acc_v, deg_s.at[s])
        plsc.subcore_barrier()
        base = c * (NS * SL) + s * SL
        for r in range(NS):
            pltpu.sync_copy(deg_s.at[r, pl.ds(base, SL)], red_v.at[r])
        for i in range(SL // 16):
            tot = red_v[0, pl.ds(i * 16, 16)]
            for r in range(1, NS):
                tot = tot + red_v[r, pl.ds(i * 16, 16)]
            x = tot + 1.0
            iv = plsc.bitcast(x, jnp.int32)
            y = plsc.bitcast(
                jnp.int32(0x5F3759DF) - jnp.right_shift(iv, 1),
                jnp.float32)
            for _ in range(3):
                y = y * (1.5 - 0.5 * x * y * y)
            dinv_v[pl.ds(i * 16, 16)] = y
        pltpu.sync_copy(dinv_v, out_hbm.at[pl.ds(base, SL)])

    return k(ei3, tail_dst, zdeg)


def _sc_agg(g, ei3, tail_src, tail_dst, zrows):
    mesh = plsc.VectorSubcoreMesh(core_axis_name="c", subcore_axis_name="s")

    @functools.partial(
        pl.kernel,
        out_type=jax.ShapeDtypeStruct((NC, N, HD), jnp.bfloat16),
        mesh=mesh,
        scratch_types=[
            pltpu.VMEM((CHS, LN), jnp.int32),
            pltpu.VMEM((CHS, LN), jnp.int32),
            [pltpu.VMEM((LN, HD), jnp.bfloat16) for _ in range(NB)],
            pltpu.VMEM((125, HD), jnp.bfloat16),
            pltpu.VMEM_SHARED((ACC_ROWS, HD), jnp.bfloat16),
            pltpu.VMEM_SHARED((N, HD), jnp.bfloat16),
            [pltpu.SemaphoreType.DMA for _ in range(NB)],
            [pltpu.SemaphoreType.DMA for _ in range(NB)],
        ],
        compiler_params=_SC_PARAMS,
    )
    def k(g_hbm, ei_hbm, ts_hbm, td_hbm, z_hbm, out_hbm,
          src_v, dst_v, rows, stage_v, acc_s, tbl_s, gs, ss):
        c = lax.axis_index("c")
        s = lax.axis_index("s")
        pltpu.sync_copy(
            z_hbm, acc_s.at[pl.ds(s * (ACC_ROWS // NS), ACC_ROWS // NS)])
        pltpu.sync_copy(g_hbm.at[c, pl.ds(s * (N // NS), N // NS)],
                        tbl_s.at[pl.ds(s * (N // NS), N // NS)])
        _load_idx(ei_hbm, 0, ts_hbm, s, src_v)
        _load_idx(ei_hbm, 1, td_hbm, s, dst_v)
        plsc.subcore_barrier()
        gc = tbl_s

        for b in range(NB):
            pltpu.async_copy(gc.at[src_v.at[b]], rows[b], gs[b])

        def body(g, carry):
            j0 = g * NB
            for b in range(NB):
                j = j0 + b
                pltpu.make_async_copy(gc.at[src_v.at[j]], rows[b], gs[b]).wait()
                pltpu.async_copy(rows[b], acc_s.at[dst_v.at[j]], ss[b], add=True)

            @pl.when(g + 1 < GROUPS)
            def _():
                for b in range(NB):
                    j = j0 + b
                    pltpu.make_async_copy(
                        rows[b], acc_s.at[dst_v.at[j]], ss[b]).wait()
                    pltpu.async_copy(gc.at[src_v.at[j + NB]], rows[b], gs[b])
            return carry

        lax.fori_loop(0, GROUPS, body, 0)
        for b in range(NB):
            pltpu.make_async_copy(
                rows[b], acc_s.at[dst_v.at[CHS - NB + b]], ss[b]).wait()
        plsc.subcore_barrier()
        for i in range(5):
            r0 = s * (N // NS) + i * 125
            pltpu.sync_copy(acc_s.at[pl.ds(r0, 125)], stage_v)
            pltpu.sync_copy(stage_v, out_hbm.at[c, pl.ds(r0, 125)])

    return k(g, ei3, tail_src, tail_dst, zrows)


def _sparsemax3(s0, s1, s2):
    z1 = jnp.maximum(s0, jnp.maximum(s1, s2))
    z3 = jnp.minimum(s0, jnp.minimum(s1, s2))
    z2 = s0 + s1 + s2 - z1 - z3
    i2 = (1.0 + 2.0 * z2 > z1 + z2).astype(jnp.float32)
    i3 = (1.0 + 3.0 * z3 > z1 + z2 + z3).astype(jnp.float32)
    kf = 1.0 + i2 + i3
    tau = jnp.where(
        kf == 3.0, (z1 + z2 + z3 - 1.0) / 3.0,
        jnp.where(kf == 2.0, (z1 + z2 - 1.0) * 0.5, z1 - 1.0))
    w0 = jnp.maximum(s0 - tau, 0.0)
    w1 = jnp.maximum(s1 - tau, 0.0)
    w2 = jnp.maximum(s2 - tau, 0.0)
    return w0, w1, w2


def _proj_body(x_ref, w_ref, b_ref, dinv_ref, g0_ref):
    dinv = dinv_ref[...]
    x0 = jnp.dot(x_ref[...].astype(jnp.bfloat16),
                 w_ref[...].astype(jnp.bfloat16),
                 preferred_element_type=jnp.float32)
    x0 = jnp.maximum(x0 + b_ref[...], 0.0)
    g0 = (x0 * dinv).astype(jnp.bfloat16)
    g0_ref[0] = g0[:, :HD]
    g0_ref[1] = g0[:, HD:]


def _tc_proj(x, w, b2, dinv):
    return pl.pallas_call(
        _proj_body,
        grid=(N // BLK,),
        in_specs=[
            pl.BlockSpec((BLK, D), lambda i: (i, 0)),
            pl.BlockSpec((D, D), lambda i: (0, 0)),
            pl.BlockSpec((1, D), lambda i: (0, 0)),
            pl.BlockSpec((BLK, 1), lambda i: (i, 0)),
        ],
        out_specs=pl.BlockSpec((NC, BLK, HD), lambda i: (0, i, 0)),
        out_shape=jax.ShapeDtypeStruct((NC, N, HD), jnp.bfloat16),
    )(x, w, b2, dinv)


def _conv_compute(acc_ref, g_ref, dinv, w0, w1, w2, att):
    xagg = jnp.concatenate(
        [acc_ref[0].astype(jnp.float32) + g_ref[0].astype(jnp.float32),
         acc_ref[1].astype(jnp.float32) + g_ref[1].astype(jnp.float32)],
        axis=1) * dinv
    xb = xagg.astype(jnp.bfloat16)
    h0 = jnp.dot(xb, w0.astype(jnp.bfloat16),
                 preferred_element_type=jnp.float32)
    h1 = jnp.dot(xb, w1.astype(jnp.bfloat16),
                 preferred_element_type=jnp.float32)
    h2 = jnp.dot(xb, w2.astype(jnp.bfloat16),
                 preferred_element_type=jnp.float32)
    s0 = jnp.sum(h0 * att, axis=1, keepdims=True)
    s1 = jnp.sum(h1 * att, axis=1, keepdims=True)
    s2 = jnp.sum(h2 * att, axis=1, keepdims=True)
    b0, b1, b2 = _sparsemax3(s0, s1, s2)
    return jnp.maximum(b0 * h0 + b1 * h1 + b2 * h2, 0.0)


def _conv_body(acc_ref, g_ref, dinv_ref, w_ref, att_ref, out_ref):
    dinv = dinv_ref[...]
    x1 = _conv_compute(acc_ref, g_ref, dinv,
                       w_ref[0], w_ref[1], w_ref[2], att_ref[...])
    g1 = (x1 * dinv).astype(jnp.bfloat16)
    out_ref[0] = g1[:, :HD]
    out_ref[1] = g1[:, HD:]


def _tc_conv(acc, g, dinv, conv_w, att2):
    return pl.pallas_call(
        _conv_body,
        grid=(N // BLK,),
        in_specs=[
            pl.BlockSpec((NC, BLK, HD), lambda i: (0, i, 0)),
            pl.BlockSpec((NC, BLK, HD), lambda i: (0, i, 0)),
            pl.BlockSpec((BLK, 1), lambda i: (i, 0)),
            pl.BlockSpec((3, D, D), lambda i: (0, 0, 0)),
            pl.BlockSpec((1, D), lambda i: (0, 0)),
        ],
        out_specs=pl.BlockSpec((NC, BLK, HD), lambda i: (0, i, 0)),
        out_shape=jax.ShapeDtypeStruct((NC, N, HD), jnp.bfloat16),
    )(acc, g, dinv, conv_w, att2)


def _final_body(acc_ref, g_ref, dinv_ref, w_ref, att_ref,
                cw_ref, cb_ref, catt_ref, out_ref):
    dinv = dinv_ref[...]
    x2 = _conv_compute(acc_ref, g_ref, dinv,
                       w_ref[0], w_ref[1], w_ref[2], att_ref[...])
    catt = catt_ref[...]
    x2b = x2.astype(jnp.bfloat16)
    cw0 = cw_ref[0].astype(jnp.bfloat16)
    cw1 = cw_ref[1].astype(jnp.bfloat16)
    cw2 = cw_ref[2].astype(jnp.bfloat16)
    h0 = jnp.dot(x2b, cw0, preferred_element_type=jnp.float32) + cb_ref[0:1, :]
    h1 = jnp.dot(x2b, cw1, preferred_element_type=jnp.float32) + cb_ref[1:2, :]
    h2 = jnp.dot(x2b, cw2, preferred_element_type=jnp.float32) + cb_ref[2:3, :]
    s0 = jnp.sum(h0 * catt, axis=1, keepdims=True)
    s1 = jnp.sum(h1 * catt, axis=1, keepdims=True)
    s2 = jnp.sum(h2 * catt, axis=1, keepdims=True)
    b0, b1, b2 = _sparsemax3(s0, s1, s2)
    out_ref[...] = b0 * h0 + b1 * h1 + b2 * h2


def _tc_final(acc, g, dinv, conv_w, att2, cwp, cbp, cattp):
    return pl.pallas_call(
        _final_body,
        grid=(N // BLK,),
        in_specs=[
            pl.BlockSpec((NC, BLK, HD), lambda i: (0, i, 0)),
            pl.BlockSpec((NC, BLK, HD), lambda i: (0, i, 0)),
            pl.BlockSpec((BLK, 1), lambda i: (i, 0)),
            pl.BlockSpec((3, D, D), lambda i: (0, 0, 0)),
            pl.BlockSpec((1, D), lambda i: (0, 0)),
            pl.BlockSpec((3, D, D), lambda i: (0, 0, 0)),
            pl.BlockSpec((3, D), lambda i: (0, 0)),
            pl.BlockSpec((1, D), lambda i: (0, 0)),
        ],
        out_specs=pl.BlockSpec((BLK, D), lambda i: (i, 0)),
        out_shape=jax.ShapeDtypeStruct((N, D), jnp.float32),
    )(acc, g, dinv, conv_w, att2, cwp, cbp, cattp)


def kernel(x, edge_index, W_lin, b_lin, conv1_w, conv1_att, conv2_w,
           conv2_att, cls_w, cls_b, cls_att):
    e = edge_index.shape[1]
    ei3 = edge_index.reshape(2, e // LN, LN)
    pad = jnp.arange(EPAD - e, dtype=jnp.int32)
    tail_src = (pad % N).reshape(TAILR, LN)
    tail_dst = (TRASH + pad % (ACC_ROWS - N)).reshape(TAILR, LN)
    zdeg = jnp.zeros((DEG_ROWS,), jnp.float32)
    zrows = jnp.zeros((ACC_ROWS // NS, HD), jnp.bfloat16)

    dinv = _sc_dinv(ei3, tail_dst, zdeg)[:N].reshape(N, 1)

    g0 = _tc_proj(x, W_lin, b_lin.reshape(1, D), dinv)
    acc1 = _sc_agg(g0, ei3, tail_src, tail_dst, zrows)
    g1 = _tc_conv(acc1, g0, dinv, conv1_w, conv1_att[:, 0].reshape(1, D))
    acc2 = _sc_agg(g1, ei3, tail_src, tail_dst, zrows)

    c = cls_w.shape[2]
    cwp = jnp.zeros((3, D, D), jnp.float32).at[:, :, :c].set(cls_w)
    cbp = jnp.zeros((3, D), jnp.float32).at[:, :c].set(cls_b)
    cattp = jnp.zeros((1, D), jnp.float32).at[0, :c].set(cls_att[:, 0])
    out = _tc_final(acc2, g1, dinv, conv2_w, conv2_att[:, 0].reshape(1, D),
                    cwp, cbp, cattp)
    return out[:, :c]
